# baseline (speedup 1.0000x reference)
import jax
import jax.numpy as jnp
from jax.experimental import pallas as pl
from jax.experimental.pallas import tpu as pltpu

N_TOKENS = 8192
D = 2048
ALPHA = 1.0
BLOCK = 1024
MCHUNK = 128
RCHUNK = 16


def _delta_kernel(t_ref, m_ref, d_ref, o_ref, am_ref):
    dmat = d_ref[:]
    f = dmat[0:1, :]
    s = dmat[1:2, :]
    b = dmat[2:3, :]

    ff = jnp.sum(f * f)
    ss = jnp.sum(s * s)
    bb = jnp.sum(b * b)
    fs = jnp.sum(f * s)
    fb = jnp.sum(f * b)
    sb = jnp.sum(s * b)

    for c in range(BLOCK // MCHUNK):
        sl = pl.ds(c * MCHUNK, MCHUNK)
        dots = jax.lax.dot_general(
            t_ref[sl, :], dmat,
            dimension_numbers=(((1,), (1,)), ((), ())),
            preferred_element_type=jnp.float32,
        )
        m = m_ref[sl, :]
        af = m * (dots[:, 0:1] / ff)
        a_s = m * ((dots[:, 1:2] - af * fs) / ss)
        ab = m * ((dots[:, 2:3] - af * fb - a_s * sb) / bb - ALPHA)
        am_ref[sl, 0:1] = af
        am_ref[sl, 1:2] = a_s
        am_ref[sl, 2:3] = ab

    def body(i, carry):
        rs = pl.ds(i * RCHUNK, RCHUNK)
        af = am_ref[rs, 0:1]
        a_s = am_ref[rs, 1:2]
        ab = am_ref[rs, 2:3]
        o_ref[rs, :] = t_ref[rs, :] - af * f - a_s * s - ab * b
        return carry

    jax.lax.fori_loop(0, BLOCK // RCHUNK, body, 0)


def kernel(t_embs, token_mask, delta_front, delta_side, delta_back):
    n, d = t_embs.shape
    m = token_mask.astype(jnp.float32).reshape(n, 1)
    dmat = jnp.concatenate(
        [delta_front[None, :], delta_side[None, :], delta_back[None, :]], axis=0
    )
    grid = (n // BLOCK,)
    return pl.pallas_call(
        _delta_kernel,
        grid=grid,
        in_specs=[
            pl.BlockSpec((BLOCK, d), lambda i: (i, 0)),
            pl.BlockSpec((BLOCK, 1), lambda i: (i, 0)),
            pl.BlockSpec((3, d), lambda i: (0, 0)),
        ],
        out_specs=pl.BlockSpec((BLOCK, d), lambda i: (i, 0)),
        out_shape=jax.ShapeDtypeStruct((n, d), t_embs.dtype),
        scratch_shapes=[pltpu.VMEM((BLOCK, 128), jnp.float32)],
        compiler_params=pltpu.CompilerParams(
            dimension_semantics=("parallel",),
        ),
    )(t_embs, m, dmat)


# speedup vs baseline: 1.4724x; 1.4724x over previous
import jax
import jax.numpy as jnp
from jax.experimental import pallas as pl
from jax.experimental.pallas import tpu as pltpu

N_TOKENS = 8192
D = 2048
ALPHA = 1.0
BLOCK = 1024
CHUNK = 512


def _delta_kernel(t_ref, m_ref, d_ref, o_ref):
    dmat = d_ref[:]
    f = dmat[0:1, :]
    s = dmat[1:2, :]
    b = dmat[2:3, :]

    ff = jnp.sum(f * f)
    ss = jnp.sum(s * s)
    bb = jnp.sum(b * b)
    fs = jnp.sum(f * s)
    fb = jnp.sum(f * b)
    sb = jnp.sum(s * b)

    for c in range(BLOCK // CHUNK):
        sl = pl.ds(c * CHUNK, CHUNK)
        dots = jax.lax.dot_general(
            t_ref[sl, :], dmat,
            dimension_numbers=(((1,), (1,)), ((), ())),
            preferred_element_type=jnp.float32,
        )
        m = m_ref[sl, :]
        af = m * (dots[:, 0:1] / ff)
        a_s = m * ((dots[:, 1:2] - af * fs) / ss)
        ab = m * ((dots[:, 2:3] - af * fb - a_s * sb) / bb - ALPHA)
        o_ref[sl, :] = t_ref[sl, :] - af * f - a_s * s - ab * b


def kernel(t_embs, token_mask, delta_front, delta_side, delta_back):
    n, d = t_embs.shape
    m = token_mask.astype(jnp.float32).reshape(n, 1)
    dmat = jnp.concatenate(
        [delta_front[None, :], delta_side[None, :], delta_back[None, :]], axis=0
    )
    grid = (n // BLOCK,)
    return pl.pallas_call(
        _delta_kernel,
        grid=grid,
        in_specs=[
            pl.BlockSpec((BLOCK, d), lambda i: (i, 0)),
            pl.BlockSpec((BLOCK, 1), lambda i: (i, 0)),
            pl.BlockSpec((3, d), lambda i: (0, 0)),
        ],
        out_specs=pl.BlockSpec((BLOCK, d), lambda i: (i, 0)),
        out_shape=jax.ShapeDtypeStruct((n, d), t_embs.dtype),
        compiler_params=pltpu.CompilerParams(
            dimension_semantics=("parallel",),
        ),
    )(t_embs, m, dmat)


# speedup vs baseline: 1.4934x; 1.0143x over previous
import jax
import jax.numpy as jnp
from jax.experimental import pallas as pl
from jax.experimental.pallas import tpu as pltpu

N_TOKENS = 8192
D = 2048
ALPHA = 1.0
BLOCK = 1024
CHUNK = 1024


def _delta_kernel(t_ref, m_ref, d_ref, o_ref):
    dmat = d_ref[:]
    f = dmat[0:1, :]
    s = dmat[1:2, :]
    b = dmat[2:3, :]

    ff = jnp.sum(f * f)
    ss = jnp.sum(s * s)
    bb = jnp.sum(b * b)
    fs = jnp.sum(f * s)
    fb = jnp.sum(f * b)
    sb = jnp.sum(s * b)

    for c in range(BLOCK // CHUNK):
        sl = pl.ds(c * CHUNK, CHUNK)
        dots = jax.lax.dot_general(
            t_ref[sl, :], dmat,
            dimension_numbers=(((1,), (1,)), ((), ())),
            preferred_element_type=jnp.float32,
        )
        m = m_ref[sl, :]
        af = m * (dots[:, 0:1] / ff)
        a_s = m * ((dots[:, 1:2] - af * fs) / ss)
        ab = m * ((dots[:, 2:3] - af * fb - a_s * sb) / bb - ALPHA)
        o_ref[sl, :] = t_ref[sl, :] - af * f - a_s * s - ab * b


def kernel(t_embs, token_mask, delta_front, delta_side, delta_back):
    n, d = t_embs.shape
    m = token_mask.astype(jnp.float32).reshape(n, 1)
    dmat = jnp.concatenate(
        [delta_front[None, :], delta_side[None, :], delta_back[None, :]], axis=0
    )
    grid = (n // BLOCK,)
    return pl.pallas_call(
        _delta_kernel,
        grid=grid,
        in_specs=[
            pl.BlockSpec((BLOCK, d), lambda i: (i, 0)),
            pl.BlockSpec((BLOCK, 1), lambda i: (i, 0)),
            pl.BlockSpec((3, d), lambda i: (0, 0)),
        ],
        out_specs=pl.BlockSpec((BLOCK, d), lambda i: (i, 0)),
        out_shape=jax.ShapeDtypeStruct((n, d), t_embs.dtype),
        compiler_params=pltpu.CompilerParams(
            dimension_semantics=("parallel",),
        ),
    )(t_embs, m, dmat)
